# Initial kernel scaffold; baseline (speedup 1.0000x reference)
import jax
import jax.numpy as jnp
from jax.experimental import pallas as pl


def kernel(multi_input, edge_index_list, lin_W0, lin_b0, lin_g, lin_be, lin_W1, lin_b1, c0_W0, c0_b0, c0_g, c0_be, c0_W1, c0_b1, eps0, c1_W0, c1_b0, c1_g, c1_be, c1_W1, c1_b1, eps1):
    raise NotImplementedError("write your pallas kernel here")


# speedup vs baseline: 4.0689x; 4.0689x over previous
import functools
import math

import jax
import jax.numpy as jnp
from jax import lax
from jax.experimental import pallas as pl
from jax.experimental.pallas import tpu as pltpu, tpu_sc as plsc

N, D, E = 10000, 128, 320000
BN_EPS = 1e-5

NS = 16
EPT = E // NS
CHUNK = 80
NCHUNK = EPT // CHUNK
STRIPE = 632
LAST_STRIPE = N - (NS - 1) * STRIPE


def _make_sc_aggregate():
    mesh = plsc.VectorSubcoreMesh(core_axis_name="c", subcore_axis_name="s")

    @functools.partial(
        pl.kernel,
        mesh=mesh,
        out_type=[
            jax.ShapeDtypeStruct((N, D), jnp.float32),
            jax.ShapeDtypeStruct((N, D), jnp.float32),
        ],
        scratch_types=[
            pltpu.VMEM_SHARED((N, D), jnp.float32),
            pltpu.VMEM((CHUNK,), jnp.int32),
            pltpu.VMEM((CHUNK,), jnp.int32),
            pltpu.VMEM((CHUNK, D), jnp.float32),
            pltpu.SemaphoreType.DMA,
        ],
    )
    def sc_aggr(x1, x2, src0, dst0, src1, dst1, zeros, aggr0, aggr1,
                accum, sidx, didx, rows, sem):
        cid = lax.axis_index("c")
        sid = lax.axis_index("s")
        r0 = pl.multiple_of(sid * STRIPE, 8)

        def stripe_copy(src_ref, dst_ref):
            @pl.when(sid < NS - 1)
            def _():
                pltpu.sync_copy(src_ref.at[pl.ds(r0, STRIPE)],
                                dst_ref.at[pl.ds(r0, STRIPE)])

            @pl.when(sid == NS - 1)
            def _():
                pltpu.sync_copy(src_ref.at[pl.ds((NS - 1) * STRIPE, LAST_STRIPE)],
                                dst_ref.at[pl.ds((NS - 1) * STRIPE, LAST_STRIPE)])

        stripe_copy(zeros, accum)
        plsc.subcore_barrier()

        ebase = pl.multiple_of(sid * EPT, 8)

        def run(x_hbm, src_hbm, dst_hbm):
            def body(j, carry):
                b = pl.multiple_of(ebase + j * CHUNK, 8)
                pltpu.sync_copy(src_hbm.at[pl.ds(b, CHUNK)], sidx)
                pltpu.async_copy(x_hbm.at[sidx], rows, sem).wait()
                pltpu.sync_copy(dst_hbm.at[pl.ds(b, CHUNK)], didx)
                pltpu.sync_copy(rows, accum.at[didx], add=True)
                return carry
            lax.fori_loop(0, NCHUNK, body, 0)

        @pl.when(cid == 0)
        def _():
            run(x1, src0, dst0)

        @pl.when(cid == 1)
        def _():
            run(x2, src1, dst1)

        plsc.subcore_barrier()

        @pl.when(cid == 0)
        def _():
            stripe_copy(accum, aggr0)

        @pl.when(cid == 1)
        def _():
            stripe_copy(accum, aggr1)

    return sc_aggr


_sc_aggregate = _make_sc_aggregate()

_BLK = 1000


def _tc_body(x0, x1, a0, x2, a1,
             wl0, bl0, sl, bel, wl1, bl1,
             w00, b00, s0, be0, w01, b01,
             w10, b10, s1, be1, w11, b11,
             e0, e1, out):
    def mlp(h, w0, b0, s, be, w1, b1):
        h = jnp.dot(h, w0[...], preferred_element_type=jnp.float32) + b0[...]
        h = h * s[...] + be[...]
        h = jnp.maximum(h, 0.0)
        return jnp.dot(h, w1[...], preferred_element_type=jnp.float32) + b1[...]

    acc = mlp(x0[...], wl0, bl0, sl, bel, wl1, bl1)
    acc = acc + mlp(e0[...] * x1[...] + a0[...], w00, b00, s0, be0, w01, b01)
    acc = acc + mlp(e1[...] * x2[...] + a1[...], w10, b10, s1, be1, w11, b11)
    out[...] = acc


def _tc_combine(x0, x1, a0, x2, a1, mats, eps_rows):
    row_spec = pl.BlockSpec((_BLK, D), lambda i: (i, 0))
    mat_spec = pl.BlockSpec((D, D), lambda i: (0, 0))
    vec_spec = pl.BlockSpec((1, D), lambda i: (0, 0))
    in_specs = ([row_spec] * 5
                + [mat_spec, vec_spec, vec_spec, vec_spec, mat_spec, vec_spec] * 3
                + [vec_spec] * 2)
    return pl.pallas_call(
        _tc_body,
        grid=(N // _BLK,),
        in_specs=in_specs,
        out_specs=pl.BlockSpec((_BLK, D), lambda i: (i, 0)),
        out_shape=jax.ShapeDtypeStruct((N, D), jnp.float32),
    )(x0, x1, a0, x2, a1, *mats, *eps_rows)


def kernel(multi_input, edge_index_list, lin_W0, lin_b0, lin_g, lin_be,
           lin_W1, lin_b1, c0_W0, c0_b0, c0_g, c0_be, c0_W1, c0_b1, eps0,
           c1_W0, c1_b0, c1_g, c1_be, c1_W1, c1_b1, eps1):
    x0 = multi_input[0]
    x1 = multi_input[1]
    x2 = multi_input[2]
    src0 = edge_index_list[0, 0]
    dst0 = edge_index_list[0, 1]
    src1 = edge_index_list[1, 0]
    dst1 = edge_index_list[1, 1]
    zeros = jnp.zeros((N, D), jnp.float32)

    aggr0, aggr1 = _sc_aggregate(x1, x2, src0, dst0, src1, dst1, zeros)

    bn_c = 1.0 / math.sqrt(1.0 + BN_EPS)
    row = lambda v: v.reshape(1, D)
    mats = [lin_W0, row(lin_b0), row(lin_g * bn_c), row(lin_be), lin_W1, row(lin_b1),
            c0_W0, row(c0_b0), row(c0_g * bn_c), row(c0_be), c0_W1, row(c0_b1),
            c1_W0, row(c1_b0), row(c1_g * bn_c), row(c1_be), c1_W1, row(c1_b1)]
    e0 = jnp.full((1, D), 1.0, jnp.float32) * (1.0 + eps0)
    e1 = jnp.full((1, D), 1.0, jnp.float32) * (1.0 + eps1)

    return _tc_combine(x0, x1, aggr0, x2, aggr1, mats, [e0, e1])


# speedup vs baseline: 11.1169x; 2.7322x over previous
import functools
import math

import jax
import jax.numpy as jnp
from jax import lax
from jax.experimental import pallas as pl
from jax.experimental.pallas import tpu as pltpu, tpu_sc as plsc

N, D, E = 10000, 128, 320000
BN_EPS = 1e-5

NS = 16
EPT = E // NS
CHUNK = 80
NCHUNK = EPT // CHUNK
NBUF = 4
NVISIT = (NCHUNK + 2) // NBUF
STRIPE = 632
LAST_STRIPE = N - (NS - 1) * STRIPE


def _make_sc_aggregate():
    mesh = plsc.VectorSubcoreMesh(core_axis_name="c", subcore_axis_name="s")

    @functools.partial(
        pl.kernel,
        mesh=mesh,
        out_type=[
            jax.ShapeDtypeStruct((N, D), jnp.float32),
            jax.ShapeDtypeStruct((N, D), jnp.float32),
        ],
        scratch_types=(
            [pltpu.VMEM_SHARED((N, D), jnp.float32)]
            + [pltpu.VMEM((CHUNK, D), jnp.float32)] * NBUF
            + [pltpu.VMEM((CHUNK,), jnp.int32)] * (2 * NBUF)
            + [pltpu.SemaphoreType.DMA] * (3 * NBUF)
        ),
    )
    def sc_aggr(x1, x2, src0, dst0, src1, dst1, zeros, aggr0, aggr1,
                accum, *bufs):
        rows = bufs[:NBUF]
        sidx = bufs[NBUF:2 * NBUF]
        didx = bufs[2 * NBUF:3 * NBUF]
        gsem = bufs[3 * NBUF:4 * NBUF]
        ssem = bufs[4 * NBUF:5 * NBUF]
        isem = bufs[5 * NBUF:6 * NBUF]
        cid = lax.axis_index("c")
        sid = lax.axis_index("s")
        r0 = pl.multiple_of(sid * STRIPE, 8)

        def stripe_copy(src_ref, dst_ref):
            @pl.when(sid < NS - 1)
            def _():
                pltpu.sync_copy(src_ref.at[pl.ds(r0, STRIPE)],
                                dst_ref.at[pl.ds(r0, STRIPE)])

            @pl.when(sid == NS - 1)
            def _():
                pltpu.sync_copy(src_ref.at[pl.ds((NS - 1) * STRIPE, LAST_STRIPE)],
                                dst_ref.at[pl.ds((NS - 1) * STRIPE, LAST_STRIPE)])

        stripe_copy(zeros, accum)
        plsc.subcore_barrier()

        ebase = pl.multiple_of(sid * EPT, 8)

        def run(x_hbm, src_hbm, dst_hbm):
            def wait_scatter(b):
                pltpu.make_async_copy(rows[b], accum.at[didx[b]],
                                      ssem[b]).wait()

            def visit(j, u):
                b2, b1, b0 = u, (u + 3) % NBUF, (u + 2) % NBUF

                @pl.when(j >= 2)
                def _():
                    wait_scatter(b2)

                @pl.when(j + 2 < NCHUNK)
                def _():
                    off = pl.multiple_of(ebase + (j + 2) * CHUNK, 8)
                    pltpu.async_copy(src_hbm.at[pl.ds(off, CHUNK)],
                                     sidx[b2], isem[b2])
                    pltpu.async_copy(dst_hbm.at[pl.ds(off, CHUNK)],
                                     didx[b2], isem[b2])

                @pl.when((j + 1 >= 0) & (j + 1 < NCHUNK))
                def _():
                    pltpu.make_async_copy(src_hbm.at[pl.ds(0, CHUNK)],
                                          sidx[b1], isem[b1]).wait()
                    pltpu.make_async_copy(dst_hbm.at[pl.ds(0, CHUNK)],
                                          didx[b1], isem[b1]).wait()
                    pltpu.async_copy(x_hbm.at[sidx[b1]], rows[b1], gsem[b1])

                @pl.when(j >= 0)
                def _():
                    pltpu.make_async_copy(x_hbm.at[sidx[b0]], rows[b0],
                                          gsem[b0]).wait()
                    pltpu.async_copy(rows[b0], accum.at[didx[b0]],
                                     ssem[b0], add=True)

            def round_body(g, carry):
                for u in range(NBUF):
                    visit(NBUF * g + u - 2, u)
                return carry

            lax.fori_loop(0, NVISIT, round_body, 0)
            wait_scatter((NCHUNK - 2) % NBUF)
            wait_scatter((NCHUNK - 1) % NBUF)

        @pl.when(cid == 0)
        def _():
            run(x1, src0, dst0)

        @pl.when(cid == 1)
        def _():
            run(x2, src1, dst1)

        plsc.subcore_barrier()

        @pl.when(cid == 0)
        def _():
            stripe_copy(accum, aggr0)

        @pl.when(cid == 1)
        def _():
            stripe_copy(accum, aggr1)

    return sc_aggr


_sc_aggregate = _make_sc_aggregate()

_BLK = 1000


def _tc_body(x0, x1, a0, x2, a1,
             wl0, bl0, sl, bel, wl1, bl1,
             w00, b00, s0, be0, w01, b01,
             w10, b10, s1, be1, w11, b11,
             e0, e1, out):
    def mlp(h, w0, b0, s, be, w1, b1):
        h = jnp.dot(h, w0[...], preferred_element_type=jnp.float32) + b0[...]
        h = h * s[...] + be[...]
        h = jnp.maximum(h, 0.0)
        return jnp.dot(h, w1[...], preferred_element_type=jnp.float32) + b1[...]

    acc = mlp(x0[...], wl0, bl0, sl, bel, wl1, bl1)
    acc = acc + mlp(e0[...] * x1[...] + a0[...], w00, b00, s0, be0, w01, b01)
    acc = acc + mlp(e1[...] * x2[...] + a1[...], w10, b10, s1, be1, w11, b11)
    out[...] = acc


def _tc_combine(x0, x1, a0, x2, a1, mats, eps_rows):
    row_spec = pl.BlockSpec((_BLK, D), lambda i: (i, 0))
    mat_spec = pl.BlockSpec((D, D), lambda i: (0, 0))
    vec_spec = pl.BlockSpec((1, D), lambda i: (0, 0))
    in_specs = ([row_spec] * 5
                + [mat_spec, vec_spec, vec_spec, vec_spec, mat_spec, vec_spec] * 3
                + [vec_spec] * 2)
    return pl.pallas_call(
        _tc_body,
        grid=(N // _BLK,),
        in_specs=in_specs,
        out_specs=pl.BlockSpec((_BLK, D), lambda i: (i, 0)),
        out_shape=jax.ShapeDtypeStruct((N, D), jnp.float32),
    )(x0, x1, a0, x2, a1, *mats, *eps_rows)


def kernel(multi_input, edge_index_list, lin_W0, lin_b0, lin_g, lin_be,
           lin_W1, lin_b1, c0_W0, c0_b0, c0_g, c0_be, c0_W1, c0_b1, eps0,
           c1_W0, c1_b0, c1_g, c1_be, c1_W1, c1_b1, eps1):
    x0 = multi_input[0]
    x1 = multi_input[1]
    x2 = multi_input[2]
    src0 = edge_index_list[0, 0]
    dst0 = edge_index_list[0, 1]
    src1 = edge_index_list[1, 0]
    dst1 = edge_index_list[1, 1]
    zeros = jnp.zeros((N, D), jnp.float32)

    aggr0, aggr1 = _sc_aggregate(x1, x2, src0, dst0, src1, dst1, zeros)

    bn_c = 1.0 / math.sqrt(1.0 + BN_EPS)
    row = lambda v: v.reshape(1, D)
    mats = [lin_W0, row(lin_b0), row(lin_g * bn_c), row(lin_be), lin_W1, row(lin_b1),
            c0_W0, row(c0_b0), row(c0_g * bn_c), row(c0_be), c0_W1, row(c0_b1),
            c1_W0, row(c1_b0), row(c1_g * bn_c), row(c1_be), c1_W1, row(c1_b1)]
    e0 = jnp.full((1, D), 1.0, jnp.float32) * (1.0 + eps0)
    e1 = jnp.full((1, D), 1.0, jnp.float32) * (1.0 + eps1)

    return _tc_combine(x0, x1, aggr0, x2, aggr1, mats, [e0, e1])
